# baseline (speedup 1.0000x reference)
import functools

import jax
import jax.numpy as jnp
from jax import lax
from jax.experimental import pallas as pl
from jax.experimental.pallas import tpu as pltpu
from jax.experimental.pallas import tpu_sc as plsc

F32 = jnp.float32
_RSQRT_MAGIC = 0x5F3759DF


def _silu(x):
    return x * jax.nn.sigmoid(x)


def _ln(x, g, b):
    m = jnp.mean(x, axis=-1, keepdims=True)
    v = jnp.var(x, axis=-1, keepdims=True)
    return (x - m) * lax.rsqrt(v + 1e-5) * g + b


def _mm(a, b):
    return jnp.dot(a, b, preferred_element_type=F32)


def _in_body(h, wi1, bi1, wi2, bi2, ng, nb, w1r, w1c, xn_o, a_o, b_o):
    x = _silu(_mm(h[...], wi1[...]) + bi1[...])
    x = _mm(x, wi2[...]) + bi2[...]
    xn = _ln(x, ng[...], nb[...])
    xn_o[...] = xn
    a_o[...] = _mm(xn, w1r[...])
    b_o[...] = _mm(xn, w1c[...])


def _mid_body(xn, s2, deg, ew2, eb2, nw1a, nw1b, nb1, nlg, nlb, nw2, nb2,
              ng, nb, w1r, w1c, xn_o, a_o, b_o):
    xn_v = xn[...]
    s = s2[0] + s2[1]
    degc = deg[0, :, 0:1] + deg[1, :, 0:1]
    agg = _mm(s, ew2[...]) + degc * eb2[...]
    a = _mm(xn_v, nw1a[...]) + _mm(agg, nw1b[...]) + nb1[...]
    a = _silu(_ln(a, nlg[...], nlb[...]))
    a = _mm(a, nw2[...]) + nb2[...]
    x_new = xn_v + a
    xn2 = _ln(x_new, ng[...], nb[...])
    xn_o[...] = xn2
    a_o[...] = _mm(xn2, w1r[...])
    b_o[...] = _mm(xn2, w1c[...])


def _fin_body(xn, s2, deg, ew2, eb2, nw1a, nw1b, nb1, nlg, nlb, nw2, nb2,
              ow1, ob1, ow2, ob2, out_o):
    xn_v = xn[...]
    s = s2[0] + s2[1]
    degc = deg[0, :, 0:1] + deg[1, :, 0:1]
    agg = _mm(s, ew2[...]) + degc * eb2[...]
    a = _mm(xn_v, nw1a[...]) + _mm(agg, nw1b[...]) + nb1[...]
    a = _silu(_ln(a, nlg[...], nlb[...]))
    a = _mm(a, nw2[...]) + nb2[...]
    x_new = xn_v + a
    o = _silu(_mm(x_new, ow1[...]) + ob1[...])
    out_o[...] = _mm(o, ow2[...]) + ob2[...]


def _c_body(ea, w1e, b1, c_o):
    nl = w1e.shape[0]
    ea_v = ea[...]
    for l in range(nl):
        c_o[l] = _mm(ea_v, w1e[l]) + b1[l]


def _full_spec(x):
    r = x.ndim
    return pl.BlockSpec(x.shape, lambda i, _r=r: (0,) * _r)


def _row_spec(bn, shape):
    rest = shape[1:]
    if len(shape) == 3:
        return pl.BlockSpec((shape[0], bn) + shape[2:], lambda i: (0, i, 0))
    return pl.BlockSpec((bn,) + rest, lambda i: (i,) + (0,) * len(rest))


def _tc_call(body, n_rows, bn, row_in_idx, ins, outs):
    grid = (n_rows // bn,)
    in_specs = []
    for i, x in enumerate(ins):
        in_specs.append(_row_spec(bn, x.shape) if i in row_in_idx else _full_spec(x))
    out_specs = [_row_spec(bn, o.shape) for o in outs]
    out_shape = [jax.ShapeDtypeStruct(o.shape, o.dtype) for o in outs]
    return pl.pallas_call(
        body,
        grid=grid,
        in_specs=in_specs,
        out_specs=out_specs if len(outs) > 1 else out_specs[0],
        out_shape=out_shape if len(outs) > 1 else out_shape[0],
    )(*ins)


def _edge_compute(buf_a, buf_b, buf_c, buf_u, lglb_v, n_edges, hid):
    nv = hid // 16
    lg = [lglb_v[0, pl.ds(j * 16, 16)] for j in range(nv)]
    lb = [lglb_v[1, pl.ds(j * 16, 16)] for j in range(nv)]

    @plsc.parallel_loop(0, n_edges, 1, unroll=4)
    def body(e):
        vs = []
        for j in range(nv):
            sl = pl.ds(j * 16, 16)
            vs.append(buf_a[e, sl] + buf_b[e, sl] + buf_c[e, sl])

        def tree(xs):
            while len(xs) > 1:
                xs = [xs[i] + xs[i + 1] for i in range(0, len(xs) - 1, 2)] + \
                     (xs[-1:] if len(xs) % 2 else [])
            return xs[0]

        s1 = jnp.sum(tree(vs))
        s2 = jnp.sum(tree([v * v for v in vs]))
        mean = s1 * (1.0 / hid)
        var = s2 * (1.0 / hid) - mean * mean + 1e-5
        var_v = jnp.full((16,), var, F32)
        bits = plsc.bitcast(var_v, jnp.int32)
        y = plsc.bitcast(jnp.full((16,), _RSQRT_MAGIC, jnp.int32) - (bits >> 1), F32)
        for _ in range(3):
            y = y * (1.5 - 0.5 * var_v * y * y)
        mg = jnp.full((16,), mean, F32) * y
        for j in range(nv):
            u = vs[j] * y - mg
            u = u * lg[j] + lb[j]
            u = u / (1.0 + jnp.exp(-u))
            buf_u[e, pl.ds(j * 16, 16)] = u


def _make_edge_kernel(n_nodes, n_edges, hid, chunk):
    mesh = plsc.VectorSubcoreMesh(core_axis_name="c", subcore_axis_name="s")
    nc, ns = 2, 16
    epw = n_edges // (nc * ns)
    n_chunks = epw // chunk
    rpt = (n_nodes // ns) // 8 * 8
    tail = n_nodes - rpt * ns

    assert n_chunks % 2 == 0 and n_chunks >= 4

    @functools.partial(
        pl.kernel,
        mesh=mesh,
        out_type=jax.ShapeDtypeStruct((nc, n_nodes, hid), F32),
        scratch_types=[
            pltpu.VMEM((2, chunk), jnp.int32),
            pltpu.VMEM((2, chunk), jnp.int32),
            pltpu.VMEM((2, chunk, hid), F32),
            pltpu.VMEM((2, chunk, hid), F32),
            pltpu.VMEM((2, chunk, hid), F32),
            pltpu.VMEM((chunk, hid), F32),
            pltpu.VMEM((2, hid), F32),
            pltpu.VMEM_SHARED((n_nodes, hid), F32),
            pltpu.SemaphoreType.DMA,
            pltpu.SemaphoreType.DMA,
            pltpu.SemaphoreType.DMA,
            pltpu.SemaphoreType.DMA,
            pltpu.SemaphoreType.DMA,
            pltpu.SemaphoreType.DMA,
        ],
        compiler_params=pltpu.CompilerParams(needs_layout_passes=False),
    )
    def edge_k(a_hbm, b_hbm, c_hbm, row_hbm, col_hbm, lglb_hbm, zero_hbm,
               out_hbm, row_v, col_v, buf_a, buf_b, buf_c, buf_u, lglb_v,
               s_sh, sem_a0, sem_a1, sem_b0, sem_b1, sem_c0, sem_c1):
        cid = lax.axis_index("c")
        sid = lax.axis_index("s")
        base = (cid * ns + sid) * epw
        sem_a = (sem_a0, sem_a1)
        sem_b = (sem_b0, sem_b1)
        sem_c = (sem_c0, sem_c1)

        def start(slot, i):
            off = base + i * chunk
            pltpu.sync_copy(row_hbm.at[pl.ds(off, chunk)], row_v.at[slot])
            pltpu.sync_copy(col_hbm.at[pl.ds(off, chunk)], col_v.at[slot])
            pltpu.async_copy(a_hbm.at[row_v.at[slot]], buf_a.at[slot],
                             sem_a[slot])
            pltpu.async_copy(b_hbm.at[col_v.at[slot]], buf_b.at[slot],
                             sem_b[slot])
            pltpu.async_copy(c_hbm.at[pl.ds(off, chunk)], buf_c.at[slot],
                             sem_c[slot])

        def finish(slot):
            pltpu.make_async_copy(a_hbm.at[row_v.at[slot]], buf_a.at[slot],
                                  sem_a[slot]).wait()
            pltpu.make_async_copy(b_hbm.at[col_v.at[slot]], buf_b.at[slot],
                                  sem_b[slot]).wait()
            pltpu.make_async_copy(c_hbm.at[pl.ds(0, chunk)], buf_c.at[slot],
                                  sem_c[slot]).wait()
            _edge_compute(buf_a.at[slot], buf_b.at[slot], buf_c.at[slot],
                          buf_u, lglb_v, chunk, hid)
            pltpu.sync_copy(buf_u, s_sh.at[row_v.at[slot]], add=True)

        stripe = pl.ds(sid * rpt, rpt)
        pltpu.sync_copy(zero_hbm.at[stripe], s_sh.at[stripe])
        if tail:
            @pl.when(sid == ns - 1)
            def _():
                ts = pl.ds(ns * rpt, tail)
                pltpu.sync_copy(zero_hbm.at[ts], s_sh.at[ts])
        pltpu.sync_copy(lglb_hbm, lglb_v)
        plsc.subcore_barrier()

        start(0, 0)

        def pair_body(g, carry):
            i = g * 2
            start(1, i + 1)
            finish(0)
            start(0, i + 2)
            finish(1)
            return carry

        lax.fori_loop(0, n_chunks // 2 - 1, pair_body, 0)
        start(1, n_chunks - 1)
        finish(0)
        finish(1)

        plsc.subcore_barrier()
        pltpu.sync_copy(s_sh.at[stripe], out_hbm.at[cid, stripe])
        if tail:
            @pl.when(sid == ns - 1)
            def _():
                ts = pl.ds(ns * rpt, tail)
                pltpu.sync_copy(s_sh.at[ts], out_hbm.at[cid, ts])

    return edge_k


def _make_deg_kernel(n_nodes, n_edges, chunk):
    mesh = plsc.VectorSubcoreMesh(core_axis_name="c", subcore_axis_name="s")
    nc, ns = 2, 16
    epw = n_edges // (nc * ns)
    n_chunks = epw // chunk
    rpt = (n_nodes // ns) // 8 * 8
    tail = n_nodes - rpt * ns

    @functools.partial(
        pl.kernel,
        mesh=mesh,
        out_type=jax.ShapeDtypeStruct((nc, n_nodes, 16), F32),
        scratch_types=[
            pltpu.VMEM((chunk,), jnp.int32),
            pltpu.VMEM((chunk, 16), F32),
            pltpu.VMEM_SHARED((n_nodes, 16), F32),
        ],
        compiler_params=pltpu.CompilerParams(needs_layout_passes=False),
    )
    def deg_k(row_hbm, zero_hbm, out_hbm, row_v, ones_v, d_sh):
        cid = lax.axis_index("c")
        sid = lax.axis_index("s")
        base = (cid * ns + sid) * epw
        stripe = pl.ds(sid * rpt, rpt)
        pltpu.sync_copy(zero_hbm.at[stripe], d_sh.at[stripe])
        if tail:
            @pl.when(sid == ns - 1)
            def _():
                ts = pl.ds(ns * rpt, tail)
                pltpu.sync_copy(zero_hbm.at[ts], d_sh.at[ts])

        def fill(e, carry):
            ones_v[e, :] = jnp.full((16,), 1.0, F32)
            return carry

        lax.fori_loop(0, chunk, fill, 0)
        plsc.subcore_barrier()

        def chunk_body(i, carry):
            off = base + i * chunk
            pltpu.sync_copy(row_hbm.at[pl.ds(off, chunk)], row_v)
            pltpu.sync_copy(ones_v, d_sh.at[row_v], add=True)
            return carry

        lax.fori_loop(0, n_chunks, chunk_body, 0)
        plsc.subcore_barrier()
        pltpu.sync_copy(d_sh.at[stripe], out_hbm.at[cid, stripe])
        if tail:
            @pl.when(sid == ns - 1)
            def _():
                ts = pl.ds(ns * rpt, tail)
                pltpu.sync_copy(d_sh.at[ts], out_hbm.at[cid, ts])

    return deg_k


def kernel(h, edges, edge_attr, params):
    p = params
    n, d = h.shape
    e = edges.shape[1]
    hid = p["emb_in"]["W1"].shape[1]
    nl = len(p["layers"])
    row = edges[0]
    col = edges[1]

    def r1(v):
        return v.reshape(1, -1)

    bn = 2000
    chunk = 40

    zero_nh = jnp.zeros((n, hid), F32)
    zero_n16 = jnp.zeros((n, 16), F32)

    w1e_all = jnp.stack([lyr["e_W1"][2 * hid:] for lyr in p["layers"]])
    b1_all = jnp.stack([r1(lyr["e_b1"]) for lyr in p["layers"]])

    c_all = _tc_call(_c_body, e, 4000, (0,), [edge_attr, w1e_all, b1_all],
                     [jax.ShapeDtypeStruct((nl, e, hid), F32)])

    deg_k = _make_deg_kernel(n, e, chunk)
    deg16 = deg_k(row, zero_n16)

    lyr0 = p["layers"][0]
    xn, a_t, b_t = _tc_call(
        _in_body, n, bn, (0,),
        [h, p["emb_in"]["W1"], r1(p["emb_in"]["b1"]), p["emb_in"]["W2"],
         r1(p["emb_in"]["b2"]), r1(lyr0["ng"]), r1(lyr0["nb"]),
         lyr0["e_W1"][:hid], lyr0["e_W1"][hid:2 * hid]],
        [jax.ShapeDtypeStruct((n, hid), F32)] * 3)

    edge_k = _make_edge_kernel(n, e, hid, chunk)

    for l in range(nl):
        lyr = p["layers"][l]
        lglb = jnp.stack([lyr["e_lg"], lyr["e_lb"]])
        s2 = edge_k(a_t, b_t, c_all[l], row, col, lglb, zero_nh)
        post = [s2, deg16, lyr["e_W2"], r1(lyr["e_b2"]),
                lyr["n_W1"][:hid], lyr["n_W1"][hid:], r1(lyr["n_b1"]),
                r1(lyr["n_lg"]), r1(lyr["n_lb"]), lyr["n_W2"], r1(lyr["n_b2"])]
        if l < nl - 1:
            nxt = p["layers"][l + 1]
            xn, a_t, b_t = _tc_call(
                _mid_body, n, bn, (0, 1, 2),
                [xn] + post + [r1(nxt["ng"]), r1(nxt["nb"]),
                               nxt["e_W1"][:hid], nxt["e_W1"][hid:2 * hid]],
                [jax.ShapeDtypeStruct((n, hid), F32)] * 3)
        else:
            out = _tc_call(
                _fin_body, n, bn, (0, 1, 2),
                [xn] + post + [p["emb_out"]["W1"], r1(p["emb_out"]["b1"]),
                               p["emb_out"]["W2"], r1(p["emb_out"]["b2"])],
                [jax.ShapeDtypeStruct((n, d), F32)])
    return out


# speedup vs baseline: 1.3803x; 1.3803x over previous
import functools

import jax
import jax.numpy as jnp
from jax import lax
from jax.experimental import pallas as pl
from jax.experimental.pallas import tpu as pltpu
from jax.experimental.pallas import tpu_sc as plsc

F32 = jnp.float32
_RSQRT_MAGIC = 0x5F3759DF


def _silu(x):
    return x * jax.nn.sigmoid(x)


def _ln(x, g, b):
    m = jnp.mean(x, axis=-1, keepdims=True)
    v = jnp.var(x, axis=-1, keepdims=True)
    return (x - m) * lax.rsqrt(v + 1e-5) * g + b


def _mm(a, b):
    return jnp.dot(a, b, preferred_element_type=F32)


def _in_body(h, wi1, bi1, wi2, bi2, ng, nb, w1r, w1c, xn_o, a_o, b_o):
    x = _silu(_mm(h[...], wi1[...]) + bi1[...])
    x = _mm(x, wi2[...]) + bi2[...]
    xn = _ln(x, ng[...], nb[...])
    xn_o[...] = xn
    a_o[...] = _mm(xn, w1r[...]).astype(a_o.dtype)
    b_o[...] = _mm(xn, w1c[...]).astype(b_o.dtype)


def _mid_body(xn, s2, deg, ew2, eb2, nw1a, nw1b, nb1, nlg, nlb, nw2, nb2,
              ng, nb, w1r, w1c, xn_o, a_o, b_o):
    xn_v = xn[...]
    s = s2[0] + s2[1]
    degc = deg[0, :, 0:1] + deg[1, :, 0:1]
    agg = _mm(s, ew2[...]) + degc * eb2[...]
    a = _mm(xn_v, nw1a[...]) + _mm(agg, nw1b[...]) + nb1[...]
    a = _silu(_ln(a, nlg[...], nlb[...]))
    a = _mm(a, nw2[...]) + nb2[...]
    x_new = xn_v + a
    xn2 = _ln(x_new, ng[...], nb[...])
    xn_o[...] = xn2
    a_o[...] = _mm(xn2, w1r[...]).astype(a_o.dtype)
    b_o[...] = _mm(xn2, w1c[...]).astype(b_o.dtype)


def _fin_body(xn, s2, deg, ew2, eb2, nw1a, nw1b, nb1, nlg, nlb, nw2, nb2,
              ow1, ob1, ow2, ob2, out_o):
    xn_v = xn[...]
    s = s2[0] + s2[1]
    degc = deg[0, :, 0:1] + deg[1, :, 0:1]
    agg = _mm(s, ew2[...]) + degc * eb2[...]
    a = _mm(xn_v, nw1a[...]) + _mm(agg, nw1b[...]) + nb1[...]
    a = _silu(_ln(a, nlg[...], nlb[...]))
    a = _mm(a, nw2[...]) + nb2[...]
    x_new = xn_v + a
    o = _silu(_mm(x_new, ow1[...]) + ob1[...])
    out_o[...] = _mm(o, ow2[...]) + ob2[...]


def _c_body(ea, w1e_ev, b1_ev, w1e_od, b1_od, c_o):
    nl = w1e_ev.shape[0]
    ea_v = ea[...]
    for l in range(nl):
        cev = (_mm(ea_v, w1e_ev[l]) + b1_ev[l]).astype(jnp.bfloat16)
        cod = (_mm(ea_v, w1e_od[l]) + b1_od[l]).astype(jnp.bfloat16)
        lo = lax.convert_element_type(
            lax.bitcast_convert_type(cev, jnp.uint16), jnp.uint32)
        hi = lax.convert_element_type(
            lax.bitcast_convert_type(cod, jnp.uint16), jnp.uint32)
        c_o[l] = lax.bitcast_convert_type(lo | (hi << 16), jnp.int32)


def _full_spec(x):
    r = x.ndim
    return pl.BlockSpec(x.shape, lambda i, _r=r: (0,) * _r)


def _row_spec(bn, shape):
    rest = shape[1:]
    if len(shape) == 3:
        return pl.BlockSpec((shape[0], bn) + shape[2:], lambda i: (0, i, 0))
    return pl.BlockSpec((bn,) + rest, lambda i: (i,) + (0,) * len(rest))


def _tc_call(body, n_rows, bn, row_in_idx, ins, outs):
    grid = (n_rows // bn,)
    in_specs = []
    for i, x in enumerate(ins):
        in_specs.append(_row_spec(bn, x.shape) if i in row_in_idx else _full_spec(x))
    out_specs = [_row_spec(bn, o.shape) for o in outs]
    out_shape = [jax.ShapeDtypeStruct(o.shape, o.dtype) for o in outs]
    return pl.pallas_call(
        body,
        grid=grid,
        in_specs=in_specs,
        out_specs=out_specs if len(outs) > 1 else out_specs[0],
        out_shape=out_shape if len(outs) > 1 else out_shape[0],
    )(*ins)


def _edge_compute(buf_a, buf_b, buf_c, buf_u, lglb_v, n_edges, hid):
    nv = hid // 16
    lg = [lglb_v[0, pl.ds(j * 16, 16)] for j in range(nv)]
    lb = [lglb_v[1, pl.ds(j * 16, 16)] for j in range(nv)]

    @plsc.parallel_loop(0, n_edges, 1, unroll=2)
    def body(e):
        vs = []
        for j in range(hid // 32):
            cw = plsc.bitcast(buf_c[e, pl.ds(j * 16, 16)], jnp.bfloat16)
            c0, c1 = plsc.unpack(cw, format=plsc.PackFormat.INTERLEAVED,
                                 preferred_element_type=F32)
            sl0 = pl.ds(j * 32, 16)
            sl1 = pl.ds(j * 32 + 16, 16)
            vs.append(buf_a[e, sl0] + buf_b[e, sl0] + c0)
            vs.append(buf_a[e, sl1] + buf_b[e, sl1] + c1)

        def tree(xs):
            while len(xs) > 1:
                xs = [xs[i] + xs[i + 1] for i in range(0, len(xs) - 1, 2)] + \
                     (xs[-1:] if len(xs) % 2 else [])
            return xs[0]

        s1 = jnp.sum(tree(vs))
        s2 = jnp.sum(tree([v * v for v in vs]))
        mean = s1 * (1.0 / hid)
        var = s2 * (1.0 / hid) - mean * mean + 1e-5
        var_v = jnp.full((16,), var, F32)
        bits = plsc.bitcast(var_v, jnp.int32)
        y = plsc.bitcast(jnp.full((16,), _RSQRT_MAGIC, jnp.int32) - (bits >> 1), F32)
        for _ in range(3):
            y = y * (1.5 - 0.5 * var_v * y * y)
        mg = jnp.full((16,), mean, F32) * y
        for j in range(nv):
            u = vs[j] * y - mg
            u = u * lg[j] + lb[j]
            u = u / (1.0 + jnp.exp(-u))
            buf_u[e, pl.ds(j * 16, 16)] = u


def _make_edge_kernel(n_nodes, n_edges, hid, chunk):
    mesh = plsc.VectorSubcoreMesh(core_axis_name="c", subcore_axis_name="s")
    nc, ns = 2, 16
    epw = n_edges // (nc * ns)
    n_chunks = epw // chunk
    rpt = (n_nodes // ns) // 8 * 8
    tail = n_nodes - rpt * ns

    assert n_chunks % 2 == 0 and n_chunks >= 4

    @functools.partial(
        pl.kernel,
        mesh=mesh,
        out_type=jax.ShapeDtypeStruct((nc, n_nodes, hid), F32),
        scratch_types=[
            pltpu.VMEM((2, chunk), jnp.int32),
            pltpu.VMEM((2, chunk), jnp.int32),
            pltpu.VMEM((2, chunk, hid), F32),
            pltpu.VMEM((2, chunk, hid), F32),
            pltpu.VMEM((2, chunk, hid // 2), jnp.int32),
            pltpu.VMEM((chunk, hid), F32),
            pltpu.VMEM((2, hid), F32),
            pltpu.VMEM_SHARED((n_nodes, hid), F32),
            pltpu.SemaphoreType.DMA,
            pltpu.SemaphoreType.DMA,
            pltpu.SemaphoreType.DMA,
            pltpu.SemaphoreType.DMA,
            pltpu.SemaphoreType.DMA,
            pltpu.SemaphoreType.DMA,
        ],
        compiler_params=pltpu.CompilerParams(needs_layout_passes=False),
    )
    def edge_k(a_hbm, b_hbm, c_hbm, row_hbm, col_hbm, lglb_hbm, zero_hbm,
               out_hbm, row_v, col_v, buf_a, buf_b, buf_c, buf_u, lglb_v,
               s_sh, sem_a0, sem_a1, sem_b0, sem_b1, sem_c0, sem_c1):
        cid = lax.axis_index("c")
        sid = lax.axis_index("s")
        base = (cid * ns + sid) * epw
        sem_a = (sem_a0, sem_a1)
        sem_b = (sem_b0, sem_b1)
        sem_c = (sem_c0, sem_c1)

        def start(slot, i):
            off = base + i * chunk
            pltpu.sync_copy(row_hbm.at[pl.ds(off, chunk)], row_v.at[slot])
            pltpu.sync_copy(col_hbm.at[pl.ds(off, chunk)], col_v.at[slot])
            pltpu.async_copy(a_hbm.at[row_v.at[slot]], buf_a.at[slot],
                             sem_a[slot])
            pltpu.async_copy(b_hbm.at[col_v.at[slot]], buf_b.at[slot],
                             sem_b[slot])
            pltpu.async_copy(c_hbm.at[pl.ds(off, chunk)], buf_c.at[slot],
                             sem_c[slot])

        def finish(slot):
            pltpu.make_async_copy(a_hbm.at[row_v.at[slot]], buf_a.at[slot],
                                  sem_a[slot]).wait()
            pltpu.make_async_copy(b_hbm.at[col_v.at[slot]], buf_b.at[slot],
                                  sem_b[slot]).wait()
            pltpu.make_async_copy(c_hbm.at[pl.ds(0, chunk)], buf_c.at[slot],
                                  sem_c[slot]).wait()
            _edge_compute(buf_a.at[slot], buf_b.at[slot], buf_c.at[slot],
                          buf_u, lglb_v, chunk, hid)
            pltpu.sync_copy(buf_u, s_sh.at[row_v.at[slot]], add=True)

        stripe = pl.ds(sid * rpt, rpt)
        pltpu.sync_copy(zero_hbm.at[stripe], s_sh.at[stripe])
        if tail:
            @pl.when(sid == ns - 1)
            def _():
                ts = pl.ds(ns * rpt, tail)
                pltpu.sync_copy(zero_hbm.at[ts], s_sh.at[ts])
        pltpu.sync_copy(lglb_hbm, lglb_v)
        plsc.subcore_barrier()

        start(0, 0)

        def pair_body(g, carry):
            i = g * 2
            start(1, i + 1)
            finish(0)
            start(0, i + 2)
            finish(1)
            return carry

        lax.fori_loop(0, n_chunks // 2 - 1, pair_body, 0)
        start(1, n_chunks - 1)
        finish(0)
        finish(1)

        plsc.subcore_barrier()
        pltpu.sync_copy(s_sh.at[stripe], out_hbm.at[cid, stripe])
        if tail:
            @pl.when(sid == ns - 1)
            def _():
                ts = pl.ds(ns * rpt, tail)
                pltpu.sync_copy(s_sh.at[ts], out_hbm.at[cid, ts])

    return edge_k


def _make_deg_kernel(n_nodes, n_edges, chunk):
    mesh = plsc.VectorSubcoreMesh(core_axis_name="c", subcore_axis_name="s")
    nc, ns = 2, 16
    epw = n_edges // (nc * ns)
    n_chunks = epw // chunk
    rpt = (n_nodes // ns) // 8 * 8
    tail = n_nodes - rpt * ns

    @functools.partial(
        pl.kernel,
        mesh=mesh,
        out_type=jax.ShapeDtypeStruct((nc, n_nodes, 16), F32),
        scratch_types=[
            pltpu.VMEM((chunk,), jnp.int32),
            pltpu.VMEM((chunk, 16), F32),
            pltpu.VMEM_SHARED((n_nodes, 16), F32),
        ],
        compiler_params=pltpu.CompilerParams(needs_layout_passes=False),
    )
    def deg_k(row_hbm, zero_hbm, out_hbm, row_v, ones_v, d_sh):
        cid = lax.axis_index("c")
        sid = lax.axis_index("s")
        base = (cid * ns + sid) * epw
        stripe = pl.ds(sid * rpt, rpt)
        pltpu.sync_copy(zero_hbm.at[stripe], d_sh.at[stripe])
        if tail:
            @pl.when(sid == ns - 1)
            def _():
                ts = pl.ds(ns * rpt, tail)
                pltpu.sync_copy(zero_hbm.at[ts], d_sh.at[ts])

        def fill(e, carry):
            ones_v[e, :] = jnp.full((16,), 1.0, F32)
            return carry

        lax.fori_loop(0, chunk, fill, 0)
        plsc.subcore_barrier()

        def chunk_body(i, carry):
            off = base + i * chunk
            pltpu.sync_copy(row_hbm.at[pl.ds(off, chunk)], row_v)
            pltpu.sync_copy(ones_v, d_sh.at[row_v], add=True)
            return carry

        lax.fori_loop(0, n_chunks, chunk_body, 0)
        plsc.subcore_barrier()
        pltpu.sync_copy(d_sh.at[stripe], out_hbm.at[cid, stripe])
        if tail:
            @pl.when(sid == ns - 1)
            def _():
                ts = pl.ds(ns * rpt, tail)
                pltpu.sync_copy(d_sh.at[ts], out_hbm.at[cid, ts])

    return deg_k


def kernel(h, edges, edge_attr, params):
    p = params
    n, d = h.shape
    e = edges.shape[1]
    hid = p["emb_in"]["W1"].shape[1]
    nl = len(p["layers"])
    row = edges[0]
    col = edges[1]

    def r1(v):
        return v.reshape(1, -1)

    bn = 2000
    chunk = 40
    bf16 = jnp.bfloat16

    perm = [j * 32 + 2 * t + h
            for j in range(hid // 32) for h in range(2) for t in range(16)]
    perm = jnp.array(perm, jnp.int32)

    zero_nh = jnp.zeros((n, hid), F32)
    zero_n16 = jnp.zeros((n, 16), F32)

    w1e_ev = jnp.stack([lyr["e_W1"][2 * hid:, 0::2] for lyr in p["layers"]])
    w1e_od = jnp.stack([lyr["e_W1"][2 * hid:, 1::2] for lyr in p["layers"]])
    b1_ev = jnp.stack([r1(lyr["e_b1"][0::2]) for lyr in p["layers"]])
    b1_od = jnp.stack([r1(lyr["e_b1"][1::2]) for lyr in p["layers"]])

    c_i32 = _tc_call(_c_body, e, 4000, (0,),
                     [edge_attr, w1e_ev, b1_ev, w1e_od, b1_od],
                     [jax.ShapeDtypeStruct((nl, e, hid // 2), jnp.int32)])

    deg_k = _make_deg_kernel(n, e, chunk)
    deg16 = deg_k(row, zero_n16)

    lyr0 = p["layers"][0]
    xn, a_t, b_t = _tc_call(
        _in_body, n, bn, (0,),
        [h, p["emb_in"]["W1"], r1(p["emb_in"]["b1"]), p["emb_in"]["W2"],
         r1(p["emb_in"]["b2"]), r1(lyr0["ng"]), r1(lyr0["nb"]),
         lyr0["e_W1"][:hid][:, perm], lyr0["e_W1"][hid:2 * hid][:, perm]],
        [jax.ShapeDtypeStruct((n, hid), F32)] * 3)

    edge_k = _make_edge_kernel(n, e, hid, chunk)

    for l in range(nl):
        lyr = p["layers"][l]
        lglb = jnp.stack([lyr["e_lg"], lyr["e_lb"]])[:, perm]
        s2 = edge_k(a_t, b_t, c_i32[l], row, col, lglb, zero_nh)
        post = [s2, deg16, lyr["e_W2"][perm], r1(lyr["e_b2"]),
                lyr["n_W1"][:hid], lyr["n_W1"][hid:], r1(lyr["n_b1"]),
                r1(lyr["n_lg"]), r1(lyr["n_lb"]), lyr["n_W2"], r1(lyr["n_b2"])]
        if l < nl - 1:
            nxt = p["layers"][l + 1]
            xn, a_t, b_t = _tc_call(
                _mid_body, n, bn, (0, 1, 2),
                [xn] + post + [r1(nxt["ng"]), r1(nxt["nb"]),
                               nxt["e_W1"][:hid][:, perm],
                               nxt["e_W1"][hid:2 * hid][:, perm]],
                [jax.ShapeDtypeStruct((n, hid), F32)] * 3)
        else:
            out = _tc_call(
                _fin_body, n, bn, (0, 1, 2),
                [xn] + post + [p["emb_out"]["W1"], r1(p["emb_out"]["b1"]),
                               p["emb_out"]["W2"], r1(p["emb_out"]["b2"])],
                [jax.ShapeDtypeStruct((n, d), F32)])
    return out
